# baseline (speedup 1.0000x reference)
import functools

import jax
import jax.numpy as jnp
from jax import lax
from jax.experimental import pallas as pl
from jax.experimental.pallas import tpu as pltpu
from jax.experimental.pallas import tpu_sc as plsc

_NUM_CLASS = 1000
_C, _H, _W = 3, 256, 128
_BATCH = 256
_D = _C * _H * _W
_K = 16
_DC = _D // _K
_NC, _NS = 2, 16
_NW = _NC * _NS
_ROWS_PER_W = _BATCH // _NW
_CHUNKS_PER_W = _ROWS_PER_W * _K
_NGATHER = 8
_G = _CHUNKS_PER_W // _NGATHER
_LANES = 16

_mesh = plsc.VectorSubcoreMesh(core_axis_name="c", subcore_axis_name="s")


@functools.partial(
    pl.kernel,
    mesh=_mesh,
    out_type=jax.ShapeDtypeStruct((_BATCH * _K, _DC), jnp.float32),
    scratch_types=[
        pltpu.VMEM((_LANES,), jnp.int32),
        pltpu.VMEM((_CHUNKS_PER_W,), jnp.int32),
        pltpu.VMEM((_NGATHER, _DC), jnp.float32),
        pltpu.VMEM((_NGATHER, _DC), jnp.float32),
        pltpu.SemaphoreType.DMA,
        pltpu.SemaphoreType.DMA,
        pltpu.SemaphoreType.DMA,
        pltpu.SemaphoreType.DMA,
    ],
)
def _gather_rows(tbl_hbm, lab_hbm, out_hbm, lab_v, idx_v, buf0, buf1,
                 sg0, sg1, sw0, sw1):
    wid = lax.axis_index("s") * _NC + lax.axis_index("c")
    row_base = wid * _CHUNKS_PER_W
    lab_base = wid * _ROWS_PER_W

    pltpu.sync_copy(lab_hbm.at[pl.ds(lab_base, _ROWS_PER_W)],
                    lab_v.at[pl.ds(0, _ROWS_PER_W)])
    iota = lax.iota(jnp.int32, _LANES)
    labs = lab_v[...]
    for t in range(_ROWS_PER_W):
        lab = lax.gather(
            labs, jnp.full((_LANES, 1), t, jnp.int32),
            lax.GatherDimensionNumbers(offset_dims=(),
                                       collapsed_slice_dims=(0,),
                                       start_index_map=(0,)),
            slice_sizes=(1,),
            mode=lax.GatherScatterMode.PROMISE_IN_BOUNDS)
        idx_v[pl.ds(t * _LANES, _LANES)] = lab * _K + iota

    bufs = (buf0, buf1)
    sgs = (sg0, sg1)
    sws = (sw0, sw1)

    def start_gather(g):
        return pltpu.async_copy(
            tbl_hbm.at[idx_v.at[pl.ds(g * _NGATHER, _NGATHER)]],
            bufs[g % 2], sgs[g % 2])

    def start_write(g):
        return pltpu.async_copy(
            bufs[g % 2],
            out_hbm.at[pl.ds(row_base + g * _NGATHER, _NGATHER)],
            sws[g % 2])

    hw = [None] * _G
    hg = [None] * _G
    hg[0] = start_gather(0)
    for g in range(_G):
        hg[g].wait()
        if g + 1 < _G:
            if g >= 1:
                hw[g - 1].wait()
            hg[g + 1] = start_gather(g + 1)
        hw[g] = start_write(g)
    hw[_G - 2].wait()
    hw[_G - 1].wait()


def kernel(label, learnable_person_info):
    tbl = learnable_person_info.reshape(_NUM_CLASS * _K, _DC)
    out = _gather_rows(tbl, label)
    return out.reshape(_BATCH, _C, _H, _W)
